# baseline (speedup 1.0000x reference)
import functools

import jax
import jax.numpy as jnp
from jax import lax
from jax.experimental import pallas as pl
from jax.experimental.pallas import tpu as pltpu
from jax.experimental.pallas import tpu_sc as plsc

_BATCH = 16384
_SEQ = 200
_D = 32
_NC = 2
_NS = 16
_NW = _NC * _NS
_BW = _BATCH // _NW
_LANE = 128
_SUB = 8
_DT = _D // _SUB
_BT_W = _BW // _LANE
_T_STRIDE = _D * _BATCH
_DT_STRIDE = _SUB * _BATCH
_W_TILE = _BW * _D


def _emb_body(table_hbm, xt_hbm, out_hbm, idx_v, rows_v, tr_v, si, sg, so):
    wid = lax.axis_index("s") * _NC + lax.axis_index("c")
    b0 = wid * _BW
    iota = lax.iota(jnp.int32, 16)
    row_ids = [iota + (btl * _LANE + g * 16)
               for btl in range(_BT_W) for g in range(_LANE // 16)]

    def start_idx(t, s):
        pltpu.async_copy(xt_hbm.at[pl.ds(t * _BATCH + b0, _BW)], idx_v[s], si[s])

    def wait_idx(s):
        pltpu.make_async_copy(xt_hbm.at[pl.ds(0, _BW)], idx_v[s], si[s]).wait()

    def start_gather(s):
        pltpu.async_copy(table_hbm.at[idx_v[s]], rows_v[s], sg[s])

    def wait_gather(s):
        pltpu.make_async_copy(table_hbm.at[idx_v[s]], rows_v[s], sg[s]).wait()

    def start_out(t, s):
        for dt in range(_DT):
            off = t * _T_STRIDE + dt * _DT_STRIDE + wid * (_BT_W * _SUB * _LANE)
            pltpu.async_copy(
                tr_v[s].at[pl.ds(dt * (_BT_W * _SUB * _LANE), _BT_W * _SUB * _LANE)],
                out_hbm.at[pl.ds(off, _BT_W * _SUB * _LANE)], so[s])

    def wait_out(s):
        pltpu.make_async_copy(tr_v[s], out_hbm.at[pl.ds(0, _W_TILE)], so[s]).wait()

    def retile(s):
        @plsc.parallel_loop(0, _D, unroll=2)
        def col_fn(k):
            col = (iota + k) & (_D - 1)
            base_off = ((col >> 3) * (_BT_W * _SUB * _LANE)
                        + (col & (_SUB - 1)) * _LANE + iota)
            for i, rid in enumerate(row_ids):
                btl, g = divmod(i, _LANE // 16)
                vals = plsc.load_gather(rows_v[s], [rid, col])
                sidx = base_off + (btl * (_SUB * _LANE) + g * 16)
                plsc.store_scatter(tr_v[s], [sidx], vals)

    def body(t, s):
        t = jnp.int32(t)
        wait_gather(s)

        @pl.when(t + 2 < _SEQ)
        def _():
            wait_idx((s + 2) % 3)
            start_gather((s + 2) % 3)

        @pl.when(t >= 3)
        def _():
            wait_out(s)

        retile(s)
        start_out(t, s)

        @pl.when(t + 3 < _SEQ)
        def _():
            start_idx(t + 3, s)

    for s in range(3):
        start_idx(s, s)
    wait_idx(0)
    start_gather(0)
    wait_idx(1)
    start_gather(1)

    def step_fn(gstep, carry):
        for j in range(6):
            body(6 * gstep + j, j % 3)
        return carry

    lax.fori_loop(0, (_SEQ - 2) // 6, step_fn, 0)
    body(_SEQ - 2, (_SEQ - 2) % 3)
    body(_SEQ - 1, (_SEQ - 1) % 3)
    for s in range(3):
        wait_out(s)


@jax.jit
def kernel(x, emb_tim):
    xt = x.T.reshape(-1).astype(jnp.int32)
    mesh = plsc.VectorSubcoreMesh(core_axis_name="c", subcore_axis_name="s")
    run = functools.partial(
        pl.kernel,
        mesh=mesh,
        out_type=jax.ShapeDtypeStruct((_SEQ * _D * _BATCH,), jnp.float32),
        scratch_types=[
            [pltpu.VMEM((_BW,), jnp.int32) for _ in range(3)],
            [pltpu.VMEM((_BW, _D), jnp.float32) for _ in range(3)],
            [pltpu.VMEM((_W_TILE,), jnp.float32) for _ in range(3)],
            [pltpu.SemaphoreType.DMA for _ in range(3)],
            [pltpu.SemaphoreType.DMA for _ in range(3)],
            [pltpu.SemaphoreType.DMA for _ in range(3)],
        ],
        compiler_params=pltpu.CompilerParams(
            use_tc_tiling_on_sc=False, needs_layout_passes=False),
    )(_emb_body)
    flat = run(emb_tim, xt)
    return (flat.reshape(_SEQ, _DT, _BATCH // _LANE, _SUB, _LANE)
            .transpose(2, 4, 0, 1, 3)
            .reshape(_BATCH, _SEQ, _D))


# speedup vs baseline: 1.1751x; 1.1751x over previous
import functools

import jax
import jax.numpy as jnp
from jax import lax
from jax.experimental import pallas as pl
from jax.experimental.pallas import tpu as pltpu
from jax.experimental.pallas import tpu_sc as plsc

_BATCH = 16384
_SEQ = 200
_D = 32
_NC = 2
_NS = 16
_NW = _NC * _NS
_BW = _BATCH // _NW
_LANE = 128
_SUB = 8
_DT = _D // _SUB
_BT_W = _BW // _LANE
_T_STRIDE = _D * _BATCH
_DT_STRIDE = _SUB * _BATCH
_W_TILE = _BW * _D


def _emb_body(table_hbm, xt_hbm, out_hbm, idx_v, rows_v, tr_v, si, sg, so):
    wid = lax.axis_index("s") * _NC + lax.axis_index("c")
    b0 = wid * _BW
    iota = lax.iota(jnp.int32, 16)
    row_ids = [iota + (btl * _LANE + g * 16)
               for btl in range(_BT_W) for g in range(_LANE // 16)]

    def start_idx(t, s):
        pltpu.async_copy(xt_hbm.at[pl.ds(t * _BATCH + b0, _BW)], idx_v[s], si[s])

    def wait_idx(s):
        pltpu.make_async_copy(xt_hbm.at[pl.ds(0, _BW)], idx_v[s], si[s]).wait()

    def start_gather(s):
        pltpu.async_copy(table_hbm.at[idx_v[s]], rows_v[s], sg[s])

    def wait_gather(s):
        pltpu.make_async_copy(table_hbm.at[idx_v[s]], rows_v[s], sg[s]).wait()

    def start_out(t, s):
        for dt in range(_DT):
            off = t * _T_STRIDE + dt * _DT_STRIDE + wid * (_BT_W * _SUB * _LANE)
            pltpu.async_copy(
                tr_v[s].at[pl.ds(dt * (_BT_W * _SUB * _LANE), _BT_W * _SUB * _LANE)],
                out_hbm.at[pl.ds(off, _BT_W * _SUB * _LANE)], so[s])

    def wait_out(s):
        pltpu.make_async_copy(tr_v[s], out_hbm.at[pl.ds(0, _W_TILE)], so[s]).wait()

    def retile(s):
        @plsc.parallel_loop(0, _D, unroll=2)
        def col_fn(k):
            col = (iota + k) & (_D - 1)
            base_off = ((col >> 3) * (_BT_W * _SUB * _LANE)
                        + (col & (_SUB - 1)) * _LANE + iota)
            for i, rid in enumerate(row_ids):
                btl, g = divmod(i, _LANE // 16)
                vals = plsc.load_gather(rows_v[s], [rid, col])
                sidx = base_off + (btl * (_SUB * _LANE) + g * 16)
                plsc.store_scatter(tr_v[s], [sidx], vals)

    start_idx(0, 0)
    start_idx(1, 1)
    wait_idx(0)
    start_gather(0)

    def step_fn(gstep, carry):
        for s in (0, 1):
            t = 2 * gstep + s
            wait_gather(s)

            @pl.when(t + 1 < _SEQ)
            def _():
                wait_idx(s ^ 1)
                start_gather(s ^ 1)

            @pl.when(t >= 2)
            def _():
                wait_out(s)

            retile(s)
            start_out(t, s)

            @pl.when(t + 2 < _SEQ)
            def _():
                start_idx(t + 2, s)
        return carry

    lax.fori_loop(0, _SEQ // 2, step_fn, 0)
    wait_out(0)
    wait_out(1)


@jax.jit
def kernel(x, emb_tim):
    xt = x.T.reshape(-1).astype(jnp.int32)
    mesh = plsc.VectorSubcoreMesh(core_axis_name="c", subcore_axis_name="s")
    run = functools.partial(
        pl.kernel,
        mesh=mesh,
        out_type=jax.ShapeDtypeStruct((_SEQ * _D * _BATCH,), jnp.float32),
        scratch_types=[
            [pltpu.VMEM((_BW,), jnp.int32) for _ in range(2)],
            [pltpu.VMEM((_BW, _D), jnp.float32) for _ in range(2)],
            [pltpu.VMEM((_W_TILE,), jnp.float32) for _ in range(2)],
            [pltpu.SemaphoreType.DMA for _ in range(2)],
            [pltpu.SemaphoreType.DMA for _ in range(2)],
            [pltpu.SemaphoreType.DMA for _ in range(2)],
        ],
        compiler_params=pltpu.CompilerParams(
            use_tc_tiling_on_sc=False, needs_layout_passes=False),
    )(_emb_body)
    flat = run(emb_tim, xt)
    return (flat.reshape(_SEQ, _DT, _BATCH // _LANE, _SUB, _LANE)
            .transpose(2, 4, 0, 1, 3)
            .reshape(_BATCH, _SEQ, _D))


# speedup vs baseline: 1.1861x; 1.0093x over previous
import functools

import jax
import jax.numpy as jnp
from jax import lax
from jax.experimental import pallas as pl
from jax.experimental.pallas import tpu as pltpu
from jax.experimental.pallas import tpu_sc as plsc

_BATCH = 16384
_SEQ = 200
_D = 32
_NC = 2
_NS = 16
_NW = _NC * _NS
_BW = _BATCH // _NW
_LANE = 128
_SUB = 8
_DT = _D // _SUB
_BT_W = _BW // _LANE
_T_STRIDE = _D * _BATCH
_DT_STRIDE = _SUB * _BATCH
_W_TILE = _BW * _D


def _emb_body(table_hbm, xt_hbm, out_hbm, idx_v, rows_v, tr_v, si, sg, so):
    wid = lax.axis_index("s") * _NC + lax.axis_index("c")
    b0 = wid * _BW
    iota = lax.iota(jnp.int32, 16)
    row_ids = [iota + (btl * _LANE + g * 16)
               for btl in range(_BT_W) for g in range(_LANE // 16)]

    def start_idx(t, s):
        for c in range(_BT_W):
            off = (((t // _SUB) * (_BATCH // _LANE) + wid * _BT_W + c)
                   * (_SUB * _LANE) + (t % _SUB) * _LANE)
            pltpu.async_copy(xt_hbm.at[pl.ds(off, _LANE)],
                             idx_v[s].at[pl.ds(c * _LANE, _LANE)], si[s])

    def wait_idx(s):
        pltpu.make_async_copy(xt_hbm.at[pl.ds(0, _BW)], idx_v[s], si[s]).wait()

    def start_gather(s):
        pltpu.async_copy(table_hbm.at[idx_v[s]], rows_v[s], sg[s])

    def wait_gather(s):
        pltpu.make_async_copy(table_hbm.at[idx_v[s]], rows_v[s], sg[s]).wait()

    def start_out(t, s):
        for dt in range(_DT):
            off = t * _T_STRIDE + dt * _DT_STRIDE + wid * (_BT_W * _SUB * _LANE)
            pltpu.async_copy(
                tr_v[s].at[pl.ds(dt * (_BT_W * _SUB * _LANE), _BT_W * _SUB * _LANE)],
                out_hbm.at[pl.ds(off, _BT_W * _SUB * _LANE)], so[s])

    def wait_out(s):
        pltpu.make_async_copy(tr_v[s], out_hbm.at[pl.ds(0, _W_TILE)], so[s]).wait()

    def retile(s):
        @plsc.parallel_loop(0, _D, unroll=2)
        def col_fn(k):
            col = (iota + k) & (_D - 1)
            base_off = ((col >> 3) * (_BT_W * _SUB * _LANE)
                        + (col & (_SUB - 1)) * _LANE + iota)
            for i, rid in enumerate(row_ids):
                btl, g = divmod(i, _LANE // 16)
                vals = plsc.load_gather(rows_v[s], [rid, col])
                sidx = base_off + (btl * (_SUB * _LANE) + g * 16)
                plsc.store_scatter(tr_v[s], [sidx], vals)

    start_idx(0, 0)
    start_idx(1, 1)
    wait_idx(0)
    start_gather(0)

    def step_fn(gstep, carry):
        for s in (0, 1):
            t = 2 * gstep + s
            wait_gather(s)

            @pl.when(t + 1 < _SEQ)
            def _():
                wait_idx(s ^ 1)
                start_gather(s ^ 1)

            @pl.when(t >= 2)
            def _():
                wait_out(s)

            retile(s)
            start_out(t, s)

            @pl.when(t + 2 < _SEQ)
            def _():
                start_idx(t + 2, s)
        return carry

    lax.fori_loop(0, _SEQ // 2, step_fn, 0)
    wait_out(0)
    wait_out(1)


@jax.jit
def kernel(x, emb_tim):
    xt = (x.reshape(_BATCH // _LANE, _LANE, _SEQ // _SUB, _SUB)
          .transpose(2, 0, 3, 1)
          .reshape(-1).astype(jnp.int32))
    mesh = plsc.VectorSubcoreMesh(core_axis_name="c", subcore_axis_name="s")
    run = functools.partial(
        pl.kernel,
        mesh=mesh,
        out_type=jax.ShapeDtypeStruct((_SEQ * _D * _BATCH,), jnp.float32),
        scratch_types=[
            [pltpu.VMEM((_BW,), jnp.int32) for _ in range(2)],
            [pltpu.VMEM((_BW, _D), jnp.float32) for _ in range(2)],
            [pltpu.VMEM((_W_TILE,), jnp.float32) for _ in range(2)],
            [pltpu.SemaphoreType.DMA for _ in range(2)],
            [pltpu.SemaphoreType.DMA for _ in range(2)],
            [pltpu.SemaphoreType.DMA for _ in range(2)],
        ],
        compiler_params=pltpu.CompilerParams(
            use_tc_tiling_on_sc=False, needs_layout_passes=False),
    )(_emb_body)
    flat = run(emb_tim, xt)
    return (flat.reshape(_SEQ, _DT, _BATCH // _LANE, _SUB, _LANE)
            .transpose(2, 4, 0, 1, 3)
            .reshape(_BATCH, _SEQ, _D))
